# baseline (speedup 1.0000x reference)
import functools
import jax
import jax.numpy as jnp
from jax import lax
from jax.experimental import pallas as pl
from jax.experimental.pallas import tpu as pltpu
from jax.experimental.pallas import tpu_sc as plsc

_NC = 2
_NS = 16
_NW = _NC * _NS
_L = 16


def kernel(s, g):
    B, D = s.shape
    rows_per_w = B // _NW
    chunk = 8192
    chunks_per_row = D // chunk

    mesh = plsc.VectorSubcoreMesh(core_axis_name="c", subcore_axis_name="s")

    @functools.partial(
        pl.kernel,
        mesh=mesh,
        out_type=jax.ShapeDtypeStruct((B, 2, D), jnp.float32),
        scratch_types=[
            pltpu.VMEM((chunk,), jnp.float32),
            pltpu.VMEM((chunk,), jnp.float32),
            pltpu.VMEM((2, chunk), jnp.float32),
        ],
        compiler_params=pltpu.CompilerParams(needs_layout_passes=False),
    )
    def sc_k(s_hbm, g_hbm, o_hbm, s_v, g_v, o_v):
        wid = lax.axis_index("s") * _NC + lax.axis_index("c")
        row0 = wid * rows_per_w

        def zbody(i, _):
            o_v[0, pl.ds(i * _L, _L)] = jnp.zeros((_L,), jnp.float32)
            return 0
        lax.fori_loop(0, chunk // _L, zbody, 0)

        def rbody(r, _):
            b = row0 + r

            def cbody(ci, _):
                start = ci * chunk
                pltpu.sync_copy(s_hbm.at[b, pl.ds(start, chunk)], s_v)
                pltpu.sync_copy(g_hbm.at[b, pl.ds(start, chunk)], g_v)

                def vbody(i, _):
                    sl = pl.ds(i * _L, _L)
                    o_v[1, sl] = g_v[sl] * 0.5 + s_v[sl] - 0.5
                    return 0
                lax.fori_loop(0, chunk // _L, vbody, 0)

                pltpu.sync_copy(o_v, o_hbm.at[b, :, pl.ds(start, chunk)])
                return 0
            lax.fori_loop(0, chunks_per_row, cbody, 0)
            return 0
        lax.fori_loop(0, rows_per_w, rbody, 0)

    out = sc_k(s, g)
    return out.swapaxes(1, 2)
